# baseline (speedup 1.0000x reference)
import functools

import jax
import jax.numpy as jnp
from jax import lax
from jax.experimental import pallas as pl
from jax.experimental.pallas import tpu as pltpu

B, N, D = 4, 4096, 1024
BK = 512
BI = 512
BJ = 512
BM = 256


def _keys_body(x_ref, k_ref):
    xb = x_ref[...]
    k_ref[...] = xb[..., 0] + jnp.sum(xb, axis=-1)


def _rank_body(keys_ref, rank_ref, *, i_block):
    g = pl.program_id(0)
    ki = keys_ref[:, pl.ds(g * i_block, i_block)]
    i_glob = g * i_block + lax.broadcasted_iota(jnp.int32, (B, i_block, BJ), 1)

    def body(jc, acc):
        kj = keys_ref[:, pl.ds(jc * BJ, BJ)]
        j_glob = jc * BJ + lax.broadcasted_iota(jnp.int32, (B, i_block, BJ), 2)
        lt = kj[:, None, :] < ki[:, :, None]
        tie = (kj[:, None, :] == ki[:, :, None]) & (j_glob < i_glob)
        return acc + jnp.sum((lt | tie).astype(jnp.int32), axis=-1)

    rank_ref[...] = lax.fori_loop(0, N // BJ, body,
                                  jnp.zeros((B, i_block), jnp.int32))


def _invert_body(rank_ref, idx_ref, *, i_block):
    g = pl.program_id(0)
    i_glob = g * i_block + lax.broadcasted_iota(jnp.int32, (B, i_block, BJ), 1)

    def body(jc, acc):
        rj = rank_ref[:, pl.ds(jc * BJ, BJ)]
        j_glob = jc * BJ + lax.broadcasted_iota(jnp.int32, (B, i_block, BJ), 2)
        hit = (rj[:, None, :] == i_glob)
        return acc + jnp.sum(jnp.where(hit, j_glob, 0), axis=-1)

    idx_ref[...] = lax.fori_loop(0, N // BJ, body,
                                 jnp.zeros((B, i_block), jnp.int32))


def _mm_body(idx_ref, x_hbm, w_ref, b_ref, out_ref, xs_ref, sem):
    g = pl.program_id(0)
    bidx = g // (N // BM)

    def start(r, _):
        row = idx_ref[0, 0, r]
        pltpu.make_async_copy(
            x_hbm.at[bidx, pl.ds(row, 1), :],
            xs_ref.at[pl.ds(r, 1), :],
            sem,
        ).start()
        return 0

    lax.fori_loop(0, BM, start, 0)

    def drain(r, _):
        pltpu.make_async_copy(
            x_hbm.at[bidx, pl.ds(0, 1), :],
            xs_ref.at[pl.ds(r, 1), :],
            sem,
        ).wait()
        return 0

    lax.fori_loop(0, BM, drain, 0)

    acc = lax.dot_general(
        xs_ref[...], w_ref[...],
        dimension_numbers=(((1,), (1,)), ((), ())),
        preferred_element_type=jnp.float32,
    )
    out_ref[...] = (acc + b_ref[...])[None]


def _compute_row_idx(x):
    keys = pl.pallas_call(
        _keys_body,
        grid=(N // BK,),
        in_specs=[pl.BlockSpec((B, BK, D), lambda g: (0, g, 0))],
        out_specs=pl.BlockSpec((B, BK), lambda g: (0, g)),
        out_shape=jax.ShapeDtypeStruct((B, N), jnp.float32),
    )(x)

    rank = pl.pallas_call(
        functools.partial(_rank_body, i_block=BI),
        grid=(N // BI,),
        in_specs=[pl.BlockSpec((B, N), lambda g: (0, 0))],
        out_specs=pl.BlockSpec((B, BI), lambda g: (0, g)),
        out_shape=jax.ShapeDtypeStruct((B, N), jnp.int32),
    )(keys)

    row_idx = pl.pallas_call(
        functools.partial(_invert_body, i_block=BI),
        grid=(N // BI,),
        in_specs=[pl.BlockSpec((B, N), lambda g: (0, 0))],
        out_specs=pl.BlockSpec((B, BI), lambda g: (0, g)),
        out_shape=jax.ShapeDtypeStruct((B, N), jnp.int32),
    )(rank)
    return row_idx


def kernel(x, W, b):
    row_idx = _compute_row_idx(x)
    idx3 = row_idx.reshape(B * N // BM, 1, BM)

    out = pl.pallas_call(
        _mm_body,
        grid=(B * N // BM,),
        in_specs=[
            pl.BlockSpec((1, 1, BM), lambda g: (g, 0, 0),
                         memory_space=pltpu.SMEM),
            pl.BlockSpec(memory_space=pl.ANY),
            pl.BlockSpec((D, D), lambda g: (0, 0)),
            pl.BlockSpec((1, D), lambda g: (0, 0)),
        ],
        out_specs=pl.BlockSpec((1, BM, D),
                               lambda g: (g // (N // BM), g % (N // BM), 0)),
        out_shape=jax.ShapeDtypeStruct((B, N, D), jnp.float32),
        scratch_shapes=[
            pltpu.VMEM((BM, D), jnp.float32),
            pltpu.SemaphoreType.DMA,
        ],
    )(idx3, x, W, b.reshape(1, D))
    return out


# speedup vs baseline: 2.2301x; 2.2301x over previous
import functools

import jax
import jax.numpy as jnp
from jax import lax
from jax.experimental import pallas as pl
from jax.experimental.pallas import tpu as pltpu
from jax.experimental.pallas import tpu_sc as plsc

B, N, D = 4, 4096, 1024
BN = 256
BI = 512
BJ = 512

NC, NS = 2, 16
NW = NC * NS
RPT = (B * N) // NW
CH = 64


def _keys_mm_body(x_ref, w_ref, b_ref, k_ref, y_ref):
    xb = x_ref[...]
    k_ref[...] = xb[..., 0] + jnp.sum(xb, axis=-1)
    xm = xb.reshape(B * BN, D)
    acc = lax.dot_general(
        xm, w_ref[...],
        dimension_numbers=(((1,), (1,)), ((), ())),
        preferred_element_type=jnp.float32,
    )
    y_ref[...] = (acc + b_ref[...]).reshape(B, BN, D)


def _rank_body(keys_ref, dest_ref):
    g = pl.program_id(0)
    ki = keys_ref[:, pl.ds(g * BI, BI)]
    i_glob = g * BI + lax.broadcasted_iota(jnp.int32, (B, BI, BJ), 1)

    def body(jc, acc):
        kj = keys_ref[:, pl.ds(jc * BJ, BJ)]
        j_glob = jc * BJ + lax.broadcasted_iota(jnp.int32, (B, BI, BJ), 2)
        lt = kj[:, None, :] < ki[:, :, None]
        tie = (kj[:, None, :] == ki[:, :, None]) & (j_glob < i_glob)
        return acc + jnp.sum((lt | tie).astype(jnp.int32), axis=-1)

    rank = lax.fori_loop(0, N // BJ, body, jnp.zeros((B, BI), jnp.int32))
    b_base = N * lax.broadcasted_iota(jnp.int32, (B, BI), 0)
    dest_ref[...] = rank + b_base


def _sc_gather_body(y_hbm, dest_hbm, out_hbm, dest_v, src_v, rows_v, sem):
    wid = lax.axis_index("s") * NC + lax.axis_index("c")
    base = wid * RPT

    pltpu.sync_copy(dest_hbm, dest_v)

    def build(jc, carry):
        dvec = dest_v[pl.ds(jc * 16, 16)]
        jvec = jc * 16 + lax.broadcasted_iota(jnp.int32, (16,), 0)
        m = (dvec >= base) & (dvec < base + RPT)
        plsc.store_scatter(src_v, [dvec - base], jvec, mask=m)
        return carry

    lax.fori_loop(0, (B * N) // 16, build, 0)

    def chunk(c, carry):
        idx = src_v.at[pl.ds(c * CH, CH)]
        pltpu.async_copy(y_hbm.at[idx], rows_v, sem).wait()
        pltpu.sync_copy(rows_v, out_hbm.at[pl.ds(base + c * CH, CH)])
        return carry

    lax.fori_loop(0, RPT // CH, chunk, 0)


def _sc_permute(y2, dest):
    mesh = plsc.VectorSubcoreMesh(core_axis_name="c", subcore_axis_name="s")
    kfn = functools.partial(
        pl.kernel,
        mesh=mesh,
        out_type=jax.ShapeDtypeStruct((B * N, D), jnp.float32),
        scratch_types=[
            pltpu.VMEM((B * N,), jnp.int32),
            pltpu.VMEM((RPT,), jnp.int32),
            pltpu.VMEM((CH, D), jnp.float32),
            pltpu.SemaphoreType.DMA,
        ],
        compiler_params=pltpu.CompilerParams(needs_layout_passes=False),
    )(_sc_gather_body)
    return kfn(y2, dest)


def kernel(x, W, b):
    keys, y = pl.pallas_call(
        _keys_mm_body,
        grid=(N // BN,),
        in_specs=[
            pl.BlockSpec((B, BN, D), lambda g: (0, g, 0)),
            pl.BlockSpec((D, D), lambda g: (0, 0)),
            pl.BlockSpec((1, D), lambda g: (0, 0)),
        ],
        out_specs=[
            pl.BlockSpec((B, BN), lambda g: (0, g)),
            pl.BlockSpec((B, BN, D), lambda g: (0, g, 0)),
        ],
        out_shape=[
            jax.ShapeDtypeStruct((B, N), jnp.float32),
            jax.ShapeDtypeStruct((B, N, D), jnp.float32),
        ],
    )(x, W, b.reshape(1, D))

    dest = pl.pallas_call(
        _rank_body,
        grid=(N // BI,),
        in_specs=[pl.BlockSpec((B, N), lambda g: (0, 0))],
        out_specs=pl.BlockSpec((B, BI), lambda g: (0, g)),
        out_shape=jax.ShapeDtypeStruct((B, N), jnp.int32),
    )(keys)

    out2 = _sc_permute(y.reshape(B * N, D), dest.reshape(B * N))
    return out2.reshape(B, N, D)


# speedup vs baseline: 2.7658x; 1.2402x over previous
import functools

import jax
import jax.numpy as jnp
from jax import lax
from jax.experimental import pallas as pl
from jax.experimental.pallas import tpu as pltpu
from jax.experimental.pallas import tpu_sc as plsc

B, N, D = 4, 4096, 1024
BN = 256
BI = 512
BJ = 512

NC, NS = 2, 16
NW = NC * NS
RPT = (B * N) // NW
CH = 32


def _keys_mm_body(x_ref, w_ref, b_ref, k_ref, y_ref):
    xb = x_ref[...]
    k_ref[...] = xb[..., 0] + jnp.sum(xb, axis=-1)
    xm = xb.reshape(B * BN, D).astype(jnp.bfloat16)
    acc = lax.dot_general(
        xm, w_ref[...].astype(jnp.bfloat16),
        dimension_numbers=(((1,), (1,)), ((), ())),
        preferred_element_type=jnp.float32,
    )
    y_ref[...] = (acc + b_ref[...]).reshape(B, BN, D)


def _rank_body(keys_ref, dest_ref):
    g = pl.program_id(0)
    ki = keys_ref[:, pl.ds(g * BI, BI)]
    kie = ki[:, :, None]

    def below(jc, acc):
        kj = keys_ref[:, pl.ds(jc * BJ, BJ)]
        return acc + jnp.sum((kj[:, None, :] <= kie).astype(jnp.int32), -1)

    def above(jc, acc):
        kj = keys_ref[:, pl.ds(jc * BJ, BJ)]
        return acc + jnp.sum((kj[:, None, :] < kie).astype(jnp.int32), -1)

    acc = lax.fori_loop(0, g, below, jnp.zeros((B, BI), jnp.int32))
    acc = lax.fori_loop(g + 1, N // BJ, above, acc)

    kd = keys_ref[:, pl.ds(g * BJ, BJ)][:, None, :]
    tri = (lax.broadcasted_iota(jnp.int32, (B, BI, BJ), 2)
           < lax.broadcasted_iota(jnp.int32, (B, BI, BJ), 1))
    diag = (kd < kie) | ((kd == kie) & tri)
    acc = acc + jnp.sum(diag.astype(jnp.int32), -1)

    b_base = N * lax.broadcasted_iota(jnp.int32, (B, BI), 0)
    dest_ref[...] = acc + b_base


def _sc_gather_body(y_hbm, dest_hbm, out_hbm, dest_v, src_v,
                    rows_a, rows_b, sem_a, sem_b):
    wid = lax.axis_index("s") * NC + lax.axis_index("c")
    base = wid * RPT

    pltpu.sync_copy(dest_hbm, dest_v)

    def build(jc, carry):
        dvec = dest_v[pl.ds(jc * 16, 16)]
        jvec = jc * 16 + lax.broadcasted_iota(jnp.int32, (16,), 0)
        m = (dvec >= base) & (dvec < base + RPT)
        plsc.store_scatter(src_v, [dvec - base], jvec, mask=m)
        return carry

    lax.fori_loop(0, (B * N) // 16, build, 0)

    nch = RPT // CH
    bufs, sems = (rows_a, rows_b), (sem_a, sem_b)

    def gather(c):
        idx = src_v.at[pl.ds(c * CH, CH)]
        return pltpu.async_copy(y_hbm.at[idx], bufs[c % 2], sems[c % 2])

    h = gather(0)
    for c in range(nch):
        h.wait()
        if c + 1 < nch:
            h = gather(c + 1)
        pltpu.sync_copy(bufs[c % 2], out_hbm.at[pl.ds(base + c * CH, CH)])


def _sc_permute(y2, dest):
    mesh = plsc.VectorSubcoreMesh(core_axis_name="c", subcore_axis_name="s")
    kfn = functools.partial(
        pl.kernel,
        mesh=mesh,
        out_type=jax.ShapeDtypeStruct((B * N, D), jnp.float32),
        scratch_types=[
            pltpu.VMEM((B * N,), jnp.int32),
            pltpu.VMEM((RPT,), jnp.int32),
            pltpu.VMEM((CH, D), jnp.float32),
            pltpu.VMEM((CH, D), jnp.float32),
            pltpu.SemaphoreType.DMA,
            pltpu.SemaphoreType.DMA,
        ],
        compiler_params=pltpu.CompilerParams(needs_layout_passes=False),
    )(_sc_gather_body)
    return kfn(y2, dest)


def kernel(x, W, b):
    keys, y = pl.pallas_call(
        _keys_mm_body,
        grid=(N // BN,),
        in_specs=[
            pl.BlockSpec((B, BN, D), lambda g: (0, g, 0)),
            pl.BlockSpec((D, D), lambda g: (0, 0)),
            pl.BlockSpec((1, D), lambda g: (0, 0)),
        ],
        out_specs=[
            pl.BlockSpec((B, BN), lambda g: (0, g)),
            pl.BlockSpec((B, BN, D), lambda g: (0, g, 0)),
        ],
        out_shape=[
            jax.ShapeDtypeStruct((B, N), jnp.float32),
            jax.ShapeDtypeStruct((B, N, D), jnp.float32),
        ],
    )(x, W, b.reshape(1, D))

    dest = pl.pallas_call(
        _rank_body,
        grid=(N // BI,),
        in_specs=[pl.BlockSpec((B, N), lambda g: (0, 0))],
        out_specs=pl.BlockSpec((B, BI), lambda g: (0, g)),
        out_shape=jax.ShapeDtypeStruct((B, N), jnp.int32),
    )(keys)

    out2 = _sc_permute(y.reshape(B * N, D), dest.reshape(B * N))
    return out2.reshape(B, N, D)
